# Initial kernel scaffold; baseline (speedup 1.0000x reference)
import functools

import jax
import jax.numpy as jnp
from jax import lax
from jax.experimental import pallas as pl
from jax.experimental.pallas import tpu as pltpu
from jax.experimental.pallas import tpu_sc as plsc

_EMBED_DIM = 64
_CHUNK = 128
_BLOCK = 1024


def _gather_body(n_blocks, table_hbm, idx_hbm, out_hbm, idx_v, rows_v, sem):
    info = plsc.get_sparse_core_info()
    nc = info.num_cores
    wid = lax.axis_index("s") * nc + lax.axis_index("c")
    rows_per_w = n_blocks * _BLOCK
    base = wid * rows_per_w
    pltpu.sync_copy(idx_hbm.at[pl.ds(base, rows_per_w)], idx_v)

    def block(j, carry):
        copies = []
        for c in range(_BLOCK // _CHUNK):
            idx_slice = idx_v.at[pl.ds(j * _BLOCK + c * _CHUNK, _CHUNK)]
            dst = rows_v.at[pl.ds(c * _CHUNK, _CHUNK)]
            copies.append(pltpu.async_copy(table_hbm.at[idx_slice], dst, sem))
        for cp in copies:
            cp.wait()
        pltpu.sync_copy(rows_v, out_hbm.at[pl.ds(base + j * _BLOCK, _BLOCK)])
        return carry

    lax.fori_loop(0, n_blocks, block, 0)


def _sc_gather(table, idx):
    n = idx.shape[0]
    info = plsc.get_sparse_core_info()
    nw = info.num_cores * info.num_subcores
    rows_per_w = n // nw
    n_blocks = rows_per_w // _BLOCK
    mesh = plsc.VectorSubcoreMesh(core_axis_name="c", subcore_axis_name="s")
    kern = pl.kernel(
        functools.partial(_gather_body, n_blocks),
        mesh=mesh,
        out_type=jax.ShapeDtypeStruct((n, _EMBED_DIM), jnp.float32),
        scratch_types=[
            pltpu.VMEM((rows_per_w,), jnp.int32),
            pltpu.VMEM((_BLOCK, _EMBED_DIM), jnp.float32),
            pltpu.SemaphoreType.DMA,
        ],
    )
    return kern(table, idx)


def _mm_body(x_ref, wt_ref, b_ref, o_ref):
    y = jnp.dot(x_ref[...], wt_ref[...], preferred_element_type=jnp.float32)
    o_ref[...] = jnp.maximum(y + b_ref[...], 0.0)


def _tc_linear(x, wt, b2d, block_rows):
    n = x.shape[0]
    grid = (n // block_rows,)
    return pl.pallas_call(
        _mm_body,
        grid=grid,
        in_specs=[
            pl.BlockSpec((block_rows, _EMBED_DIM), lambda i: (i, 0)),
            pl.BlockSpec((_EMBED_DIM, _EMBED_DIM), lambda i: (0, 0)),
            pl.BlockSpec((1, _EMBED_DIM), lambda i: (0, 0)),
        ],
        out_specs=pl.BlockSpec((block_rows, _EMBED_DIM), lambda i: (i, 0)),
        out_shape=jax.ShapeDtypeStruct((n, _EMBED_DIM), jnp.float32),
    )(x, wt, b2d)


def kernel(element, table, W, b):
    batch, hist = element.shape
    idx = element.reshape(-1).astype(jnp.int32)
    emb = _sc_gather(table, idx)
    out = _tc_linear(emb, W.T, b.reshape(1, _EMBED_DIM), 4096)
    return out.reshape(batch, hist, _EMBED_DIM)


# speedup vs baseline: 1.1688x; 1.1688x over previous
import functools

import jax
import jax.numpy as jnp
from jax import lax
from jax.experimental import pallas as pl
from jax.experimental.pallas import tpu as pltpu
from jax.experimental.pallas import tpu_sc as plsc

_EMBED_DIM = 64
_CHUNK = 128
_BLOCK = 1024


def _gather_body(n_blocks, table_hbm, idx_hbm, out_hbm, idx_v, rows_v, sem):
    info = plsc.get_sparse_core_info()
    nc = info.num_cores
    wid = lax.axis_index("s") * nc + lax.axis_index("c")
    rows_per_w = n_blocks * _BLOCK
    base = wid * rows_per_w
    pltpu.sync_copy(idx_hbm.at[pl.ds(base, rows_per_w)], idx_v)

    def block(j, carry):
        copies = []
        for c in range(_BLOCK // _CHUNK):
            idx_slice = idx_v.at[pl.ds(j * _BLOCK + c * _CHUNK, _CHUNK)]
            dst = rows_v.at[pl.ds(c * _CHUNK, _CHUNK)]
            copies.append(pltpu.async_copy(table_hbm.at[idx_slice], dst, sem))
        for cp in copies:
            cp.wait()
        pltpu.sync_copy(rows_v, out_hbm.at[pl.ds(base + j * _BLOCK, _BLOCK)])
        return carry

    lax.fori_loop(0, n_blocks, block, 0)


def _sc_gather(table, idx):
    n = idx.shape[0]
    info = plsc.get_sparse_core_info()
    nw = info.num_cores * info.num_subcores
    rows_per_w = n // nw
    n_blocks = rows_per_w // _BLOCK
    mesh = plsc.VectorSubcoreMesh(core_axis_name="c", subcore_axis_name="s")
    kern = pl.kernel(
        functools.partial(_gather_body, n_blocks),
        mesh=mesh,
        compiler_params=pltpu.CompilerParams(use_tc_tiling_on_sc=False),
        out_type=jax.ShapeDtypeStruct((n, _EMBED_DIM), jnp.float32),
        scratch_types=[
            pltpu.VMEM((rows_per_w,), jnp.int32),
            pltpu.VMEM((_BLOCK, _EMBED_DIM), jnp.float32),
            pltpu.SemaphoreType.DMA,
        ],
    )
    return kern(table, idx)


def _mm_body(x_ref, wt_ref, b_ref, o_ref):
    y = jnp.dot(x_ref[...], wt_ref[...], preferred_element_type=jnp.float32)
    o_ref[...] = jnp.maximum(y + b_ref[...], 0.0)


def _tc_linear(x, wt, b2d, block_rows):
    n = x.shape[0]
    grid = (n // block_rows,)
    return pl.pallas_call(
        _mm_body,
        grid=grid,
        in_specs=[
            pl.BlockSpec((block_rows, _EMBED_DIM), lambda i: (i, 0)),
            pl.BlockSpec((_EMBED_DIM, _EMBED_DIM), lambda i: (0, 0)),
            pl.BlockSpec((1, _EMBED_DIM), lambda i: (0, 0)),
        ],
        out_specs=pl.BlockSpec((block_rows, _EMBED_DIM), lambda i: (i, 0)),
        out_shape=jax.ShapeDtypeStruct((n, _EMBED_DIM), jnp.float32),
    )(x, wt, b2d)


def kernel(element, table, W, b):
    batch, hist = element.shape
    idx = element.reshape(-1).astype(jnp.int32)
    emb = _sc_gather(table, idx)
    out = _tc_linear(emb, W.T, b.reshape(1, _EMBED_DIM), 4096)
    return out.reshape(batch, hist, _EMBED_DIM)


# speedup vs baseline: 1.5158x; 1.2968x over previous
import functools

import jax
import jax.numpy as jnp
from jax import lax
from jax.experimental import pallas as pl
from jax.experimental.pallas import tpu as pltpu
from jax.experimental.pallas import tpu_sc as plsc

_EMBED_DIM = 64
_HIST = 50
_CHUNK = 128
_PBLOCK = 512


def _gather_body(n_blocks, table_hbm, idx_e_hbm, idx_o_hbm, out_hbm,
                 idx_e_v, idx_o_v, rows_e_v, rows_o_v, sem):
    info = plsc.get_sparse_core_info()
    nc = info.num_cores
    wid = lax.axis_index("s") * nc + lax.axis_index("c")
    prows_per_w = n_blocks * _PBLOCK
    base = wid * prows_per_w
    pltpu.sync_copy(idx_e_hbm.at[pl.ds(base, prows_per_w)], idx_e_v)
    pltpu.sync_copy(idx_o_hbm.at[pl.ds(base, prows_per_w)], idx_o_v)

    def block(j, carry):
        copies = []
        for c in range(_PBLOCK // _CHUNK):
            sl = pl.ds(j * _PBLOCK + c * _CHUNK, _CHUNK)
            dst = pl.ds(c * _CHUNK, _CHUNK)
            copies.append(pltpu.async_copy(
                table_hbm.at[idx_e_v.at[sl]], rows_e_v.at[dst], sem))
            copies.append(pltpu.async_copy(
                table_hbm.at[idx_o_v.at[sl]], rows_o_v.at[dst], sem))
        for cp in copies:
            cp.wait()
        out_sl = pl.ds(base + j * _PBLOCK, _PBLOCK)
        pltpu.sync_copy(rows_e_v, out_hbm.at[out_sl, pl.ds(0, _EMBED_DIM)])
        pltpu.sync_copy(rows_o_v, out_hbm.at[out_sl, pl.ds(_EMBED_DIM, _EMBED_DIM)])
        return carry

    lax.fori_loop(0, n_blocks, block, 0)


def _sc_gather_packed(table, idx_e, idx_o):
    np_ = idx_e.shape[0]
    info = plsc.get_sparse_core_info()
    nw = info.num_cores * info.num_subcores
    prows_per_w = np_ // nw
    n_blocks = prows_per_w // _PBLOCK
    mesh = plsc.VectorSubcoreMesh(core_axis_name="c", subcore_axis_name="s")
    kern = pl.kernel(
        functools.partial(_gather_body, n_blocks),
        mesh=mesh,
        compiler_params=pltpu.CompilerParams(use_tc_tiling_on_sc=False),
        out_type=jax.ShapeDtypeStruct((np_, 2 * _EMBED_DIM), jnp.float32),
        scratch_types=[
            pltpu.VMEM((prows_per_w,), jnp.int32),
            pltpu.VMEM((prows_per_w,), jnp.int32),
            pltpu.VMEM((_PBLOCK, _EMBED_DIM), jnp.float32),
            pltpu.VMEM((_PBLOCK, _EMBED_DIM), jnp.float32),
            pltpu.SemaphoreType.DMA,
        ],
    )
    return kern(table, idx_e, idx_o)


def _mm_body(b_blk, x_ref, w2_ref, b2_ref, o_ref):
    y = jnp.dot(x_ref[...], w2_ref[...], preferred_element_type=jnp.float32)
    y = jnp.maximum(y + b2_ref[...], 0.0)
    for j in range(_HIST // 2):
        yj = y[j * b_blk:(j + 1) * b_blk, :]
        o_ref[:, 2 * j, :] = yj[:, :_EMBED_DIM]
        o_ref[:, 2 * j + 1, :] = yj[:, _EMBED_DIM:]


def _tc_linear(x2, w2, b2, batch, b_blk):
    grid = (batch // b_blk,)
    return pl.pallas_call(
        functools.partial(_mm_body, b_blk),
        grid=grid,
        in_specs=[
            pl.BlockSpec((b_blk * _HIST // 2, 2 * _EMBED_DIM), lambda i: (i, 0)),
            pl.BlockSpec((2 * _EMBED_DIM, 2 * _EMBED_DIM), lambda i: (0, 0)),
            pl.BlockSpec((1, 2 * _EMBED_DIM), lambda i: (0, 0)),
        ],
        out_specs=pl.BlockSpec((b_blk, _HIST, _EMBED_DIM), lambda i: (i, 0, 0)),
        out_shape=jax.ShapeDtypeStruct((batch, _HIST, _EMBED_DIM), jnp.float32),
    )(x2, w2, b2)


def kernel(element, table, W, b):
    batch, hist = element.shape
    b_blk = 256
    e4 = element.astype(jnp.int32).reshape(batch // b_blk, b_blk, hist // 2, 2)
    e4 = e4.transpose(0, 2, 1, 3)
    idx_e = e4[..., 0].reshape(-1)
    idx_o = e4[..., 1].reshape(-1)
    emb2 = _sc_gather_packed(table, idx_e, idx_o)
    wt = W.T
    w2 = jnp.zeros((2 * _EMBED_DIM, 2 * _EMBED_DIM), wt.dtype)
    w2 = w2.at[:_EMBED_DIM, :_EMBED_DIM].set(wt)
    w2 = w2.at[_EMBED_DIM:, _EMBED_DIM:].set(wt)
    b2 = jnp.concatenate([b, b]).reshape(1, 2 * _EMBED_DIM)
    return _tc_linear(emb2, w2, b2, batch, b_blk)


# speedup vs baseline: 1.6423x; 1.0835x over previous
import functools

import jax
import jax.numpy as jnp
from jax import lax
from jax.experimental import pallas as pl
from jax.experimental.pallas import tpu as pltpu
from jax.experimental.pallas import tpu_sc as plsc

_EMBED_DIM = 64
_CHUNK = 128
_PBLOCK = 512


def _gather_body(n_blocks, table_hbm, idx_e_hbm, idx_o_hbm, out_hbm,
                 idx_e_v, idx_o_v, rows_e_v, rows_o_v, sem):
    info = plsc.get_sparse_core_info()
    nc = info.num_cores
    wid = lax.axis_index("s") * nc + lax.axis_index("c")
    prows_per_w = n_blocks * _PBLOCK
    base = wid * prows_per_w
    pltpu.sync_copy(idx_e_hbm.at[pl.ds(base, prows_per_w)], idx_e_v)
    pltpu.sync_copy(idx_o_hbm.at[pl.ds(base, prows_per_w)], idx_o_v)

    def block(j, carry):
        copies = []
        for c in range(_PBLOCK // _CHUNK):
            sl = pl.ds(j * _PBLOCK + c * _CHUNK, _CHUNK)
            dst = pl.ds(c * _CHUNK, _CHUNK)
            copies.append(pltpu.async_copy(
                table_hbm.at[idx_e_v.at[sl]], rows_e_v.at[dst], sem))
            copies.append(pltpu.async_copy(
                table_hbm.at[idx_o_v.at[sl]], rows_o_v.at[dst], sem))
        for cp in copies:
            cp.wait()
        out_sl = pl.ds(base + j * _PBLOCK, _PBLOCK)
        pltpu.sync_copy(rows_e_v, out_hbm.at[out_sl, pl.ds(0, _EMBED_DIM)])
        pltpu.sync_copy(rows_o_v, out_hbm.at[out_sl, pl.ds(_EMBED_DIM, _EMBED_DIM)])
        return carry

    lax.fori_loop(0, n_blocks, block, 0)


def _sc_gather_packed(table, idx_e, idx_o):
    np_ = idx_e.shape[0]
    info = plsc.get_sparse_core_info()
    nw = info.num_cores * info.num_subcores
    prows_per_w = np_ // nw
    n_blocks = prows_per_w // _PBLOCK
    mesh = plsc.VectorSubcoreMesh(core_axis_name="c", subcore_axis_name="s")
    kern = pl.kernel(
        functools.partial(_gather_body, n_blocks),
        mesh=mesh,
        compiler_params=pltpu.CompilerParams(use_tc_tiling_on_sc=False),
        out_type=jax.ShapeDtypeStruct((np_, 2 * _EMBED_DIM), jnp.float32),
        scratch_types=[
            pltpu.VMEM((prows_per_w,), jnp.int32),
            pltpu.VMEM((prows_per_w,), jnp.int32),
            pltpu.VMEM((_PBLOCK, _EMBED_DIM), jnp.float32),
            pltpu.VMEM((_PBLOCK, _EMBED_DIM), jnp.float32),
            pltpu.SemaphoreType.DMA,
        ],
    )
    return kern(table, idx_e, idx_o)


def _mm_body(x_ref, w2_ref, b2_ref, o_ref):
    y = jnp.dot(x_ref[...], w2_ref[...], preferred_element_type=jnp.float32)
    o_ref[...] = jnp.maximum(y + b2_ref[...], 0.0)


def _tc_linear(x2, w2, b2, block_rows):
    n = x2.shape[0]
    grid = (n // block_rows,)
    return pl.pallas_call(
        _mm_body,
        grid=grid,
        in_specs=[
            pl.BlockSpec((block_rows, 2 * _EMBED_DIM), lambda i: (i, 0)),
            pl.BlockSpec((2 * _EMBED_DIM, 2 * _EMBED_DIM), lambda i: (0, 0)),
            pl.BlockSpec((1, 2 * _EMBED_DIM), lambda i: (0, 0)),
        ],
        out_specs=pl.BlockSpec((block_rows, 2 * _EMBED_DIM), lambda i: (i, 0)),
        out_shape=jax.ShapeDtypeStruct((n, 2 * _EMBED_DIM), jnp.float32),
    )(x2, w2, b2)


def kernel(element, table, W, b):
    batch, hist = element.shape
    idx = element.reshape(-1).astype(jnp.int32)
    idx_e = idx[0::2]
    idx_o = idx[1::2]
    emb2 = _sc_gather_packed(table, idx_e, idx_o)
    wt = W.T
    w2 = jnp.zeros((2 * _EMBED_DIM, 2 * _EMBED_DIM), wt.dtype)
    w2 = w2.at[:_EMBED_DIM, :_EMBED_DIM].set(wt)
    w2 = w2.at[_EMBED_DIM:, _EMBED_DIM:].set(wt)
    b2 = jnp.concatenate([b, b]).reshape(1, 2 * _EMBED_DIM)
    y2 = _tc_linear(emb2, w2, b2, 8192)
    return y2.reshape(batch, hist, _EMBED_DIM)


# speedup vs baseline: 2.4150x; 1.4705x over previous
import functools

import jax
import jax.numpy as jnp
from jax import lax
from jax.experimental import pallas as pl
from jax.experimental.pallas import tpu as pltpu
from jax.experimental.pallas import tpu_sc as plsc

_D = 64
_PB = 4096
_CHUNK = 128
_BLOCK = 1024


def _transform_body(x_ref, wt_ref, b_ref, o_ref):
    x = x_ref[...]
    for half in range(2):
        xt = x[:, half * _PB:(half + 1) * _PB].T
        y = jnp.dot(xt, wt_ref[...], preferred_element_type=jnp.float32)
        y = jnp.maximum(y + b_ref[...], 0.0)
        o_ref[:, half * _D:(half + 1) * _D] = y


def _tc_transform(table_t, wt, b2d):
    v, n = table_t.shape
    grid_n = (n + 2 * _PB - 1) // (2 * _PB)
    return pl.pallas_call(
        _transform_body,
        grid=(grid_n,),
        in_specs=[
            pl.BlockSpec((_D, 2 * _PB), lambda i: (0, i)),
            pl.BlockSpec((_D, _D), lambda i: (0, 0)),
            pl.BlockSpec((1, _D), lambda i: (0, 0)),
        ],
        out_specs=pl.BlockSpec((_PB, 2 * _D), lambda i: (i, 0)),
        out_shape=jax.ShapeDtypeStruct((grid_n * _PB, 2 * _D), jnp.float32),
    )(table_t, wt, b2d)


def _gather_body(n_blocks, table_hbm, idx_hbm, out_hbm, idx_v, rows_v, sem):
    info = plsc.get_sparse_core_info()
    nc = info.num_cores
    wid = lax.axis_index("s") * nc + lax.axis_index("c")
    rows_per_w = n_blocks * _BLOCK
    base = wid * rows_per_w
    pltpu.sync_copy(idx_hbm.at[pl.ds(base, rows_per_w)], idx_v)

    def block(j, carry):
        copies = []
        for c in range(_BLOCK // _CHUNK):
            sl = pl.ds(j * _BLOCK + c * _CHUNK, _CHUNK)
            dst = pl.ds(c * _CHUNK, _CHUNK)
            copies.append(pltpu.async_copy(
                table_hbm.at[idx_v.at[sl]], rows_v.at[dst], sem))
        for cp in copies:
            cp.wait()
        pltpu.sync_copy(rows_v, out_hbm.at[pl.ds(base + j * _BLOCK, _BLOCK)])
        return carry

    lax.fori_loop(0, n_blocks, block, 0)


def _sc_gather(table_lin, idx):
    n = idx.shape[0]
    info = plsc.get_sparse_core_info()
    nw = info.num_cores * info.num_subcores
    rows_per_w = n // nw
    n_blocks = rows_per_w // _BLOCK
    mesh = plsc.VectorSubcoreMesh(core_axis_name="c", subcore_axis_name="s")
    kern = pl.kernel(
        functools.partial(_gather_body, n_blocks),
        mesh=mesh,
        compiler_params=pltpu.CompilerParams(use_tc_tiling_on_sc=False),
        out_type=jax.ShapeDtypeStruct((n, _D), jnp.float32),
        scratch_types=[
            pltpu.VMEM((rows_per_w,), jnp.int32),
            pltpu.VMEM((_BLOCK, _D), jnp.float32),
            pltpu.SemaphoreType.DMA,
        ],
    )
    return kern(table_lin, idx)


def kernel(element, table, W, b):
    batch, hist = element.shape
    idx = element.reshape(-1).astype(jnp.int32)
    i = idx // (2 * _PB)
    q = idx % (2 * _PB)
    idx_phys = i * (2 * _PB) + 2 * (q % _PB) + q // _PB
    p2 = _tc_transform(table.T, W.T, b.reshape(1, _D))
    p2v = p2.reshape(p2.shape[0] * 2, _D)
    y = _sc_gather(p2v, idx_phys)
    return y.reshape(batch, hist, _D)


# speedup vs baseline: 3.4990x; 1.4488x over previous
import functools

import jax
import jax.numpy as jnp
from jax import lax
from jax.experimental import pallas as pl
from jax.experimental.pallas import tpu as pltpu
from jax.experimental.pallas import tpu_sc as plsc

_D = 64
_PB = 4096
_CHUNK = 128
_BW = 512


def _transform_body(x_ref, wt_ref, b_ref, o_ref):
    x = x_ref[...]
    for half in range(2):
        xt = x[:, half * _PB:(half + 1) * _PB].T
        y = jnp.dot(xt, wt_ref[...], preferred_element_type=jnp.float32)
        y = jnp.maximum(y + b_ref[...], 0.0)
        o_ref[:, half * _D:(half + 1) * _D] = y


def _tc_transform(table_t, wt, b2d):
    v, n = table_t.shape
    grid_n = (n + 2 * _PB - 1) // (2 * _PB)
    return pl.pallas_call(
        _transform_body,
        grid=(grid_n,),
        in_specs=[
            pl.BlockSpec((_D, 2 * _PB), lambda i: (0, i)),
            pl.BlockSpec((_D, _D), lambda i: (0, 0)),
            pl.BlockSpec((1, _D), lambda i: (0, 0)),
        ],
        out_specs=pl.BlockSpec((_PB, 2 * _D), lambda i: (i, 0)),
        out_shape=jax.ShapeDtypeStruct((grid_n * _PB, 2 * _D), jnp.float32),
    )(table_t, wt, b2d)


def _gather_body(n_pairs, table_hbm, idxt_hbm, out_hbm, idx_v, rows_v, sem):
    info = plsc.get_sparse_core_info()
    nc = info.num_cores
    wid = lax.axis_index("s") * nc + lax.axis_index("c")
    b0 = wid * _BW
    pltpu.sync_copy(idxt_hbm.at[:, pl.ds(b0, _BW)], idx_v)

    def pair(j, carry):
        copies = []
        for c in range(_BW // _CHUNK):
            src_e = idx_v.at[2 * j, pl.ds(c * _CHUNK, _CHUNK)]
            src_o = idx_v.at[2 * j + 1, pl.ds(c * _CHUNK, _CHUNK)]
            copies.append(pltpu.async_copy(
                table_hbm.at[src_e], rows_v.at[pl.ds(c * _CHUNK, _CHUNK)], sem))
            copies.append(pltpu.async_copy(
                table_hbm.at[src_o],
                rows_v.at[pl.ds(_BW + c * _CHUNK, _CHUNK)], sem))
        for cp in copies:
            cp.wait()
        pltpu.sync_copy(rows_v.at[pl.ds(0, _BW)],
                        out_hbm.at[j, pl.ds(b0, _BW), pl.ds(0, _D)])
        pltpu.sync_copy(rows_v.at[pl.ds(_BW, _BW)],
                        out_hbm.at[j, pl.ds(b0, _BW), pl.ds(_D, _D)])
        return carry

    lax.fori_loop(0, n_pairs, pair, 0)


def _sc_gather(table_lin, idxt):
    hist, batch = idxt.shape
    n_pairs = hist // 2
    mesh = plsc.VectorSubcoreMesh(core_axis_name="c", subcore_axis_name="s")
    kern = pl.kernel(
        functools.partial(_gather_body, n_pairs),
        mesh=mesh,
        compiler_params=pltpu.CompilerParams(use_tc_tiling_on_sc=False),
        out_type=jax.ShapeDtypeStruct((n_pairs, batch, 2 * _D), jnp.float32),
        scratch_types=[
            pltpu.VMEM((hist, _BW), jnp.int32),
            pltpu.VMEM((2 * _BW, _D), jnp.float32),
            pltpu.SemaphoreType.DMA,
        ],
    )
    return kern(table_lin, idxt)


def _unpack_body(x_ref, o_ref):
    xt = x_ref[0].T
    o_ref[0] = xt[:_D, :]
    o_ref[1] = xt[_D:, :]


def _tc_unpack(emb3, bl):
    n_pairs, batch, _ = emb3.shape
    return pl.pallas_call(
        _unpack_body,
        grid=(n_pairs, batch // bl),
        in_specs=[pl.BlockSpec((1, bl, 2 * _D), lambda j, i: (j, i, 0))],
        out_specs=pl.BlockSpec((2, _D, bl), lambda j, i: (j, 0, i)),
        out_shape=jax.ShapeDtypeStruct((2 * n_pairs, _D, batch), jnp.float32),
    )(emb3)


def kernel(element, table, W, b):
    batch, hist = element.shape
    el = element.astype(jnp.int32)
    i = el // (2 * _PB)
    q = el % (2 * _PB)
    idx_phys = i * (2 * _PB) + 2 * (q % _PB) + q // _PB
    idxt = idx_phys.T
    p2 = _tc_transform(table.T, W.T, b.reshape(1, _D))
    p2v = p2.reshape(p2.shape[0] * 2, _D)
    emb3 = _sc_gather(p2v, idxt)
    y3 = _tc_unpack(emb3, 2048)
    return y3.transpose(2, 0, 1)


# speedup vs baseline: 4.2721x; 1.2210x over previous
import functools

import jax
import jax.numpy as jnp
from jax import lax
from jax.experimental import pallas as pl
from jax.experimental.pallas import tpu as pltpu
from jax.experimental.pallas import tpu_sc as plsc

_D = 64
_PB = 8192
_CHUNK = 128
_BW = 512


def _transform_body(x_ref, wt_ref, b_ref, o_ref):
    x = x_ref[...]
    for half in range(2):
        xt = x[:, half * _PB:(half + 1) * _PB].T
        y = jnp.dot(xt, wt_ref[...], preferred_element_type=jnp.float32)
        y = jnp.maximum(y + b_ref[...], 0.0)
        o_ref[:, half * _D:(half + 1) * _D] = y


def _tc_transform(table_t, wt, b2d):
    v, n = table_t.shape
    grid_n = (n + 2 * _PB - 1) // (2 * _PB)
    return pl.pallas_call(
        _transform_body,
        grid=(grid_n,),
        in_specs=[
            pl.BlockSpec((_D, 2 * _PB), lambda i: (0, i)),
            pl.BlockSpec((_D, _D), lambda i: (0, 0)),
            pl.BlockSpec((1, _D), lambda i: (0, 0)),
        ],
        out_specs=pl.BlockSpec((_PB, 2 * _D), lambda i: (i, 0)),
        out_shape=jax.ShapeDtypeStruct((grid_n * _PB, 2 * _D), jnp.float32),
    )(table_t, wt, b2d)


def _gather_body(n_pairs, table_hbm, idxt_hbm, out_hbm, idx_v, rows_v, sem):
    info = plsc.get_sparse_core_info()
    nc = info.num_cores
    wid = lax.axis_index("s") * nc + lax.axis_index("c")
    b0 = wid * _BW
    pltpu.sync_copy(idxt_hbm.at[:, pl.ds(b0, _BW)], idx_v)

    def pair(j, carry):
        copies = []
        for c in range(_BW // _CHUNK):
            src_e = idx_v.at[2 * j, pl.ds(c * _CHUNK, _CHUNK)]
            src_o = idx_v.at[2 * j + 1, pl.ds(c * _CHUNK, _CHUNK)]
            copies.append(pltpu.async_copy(
                table_hbm.at[src_e], rows_v.at[pl.ds(c * _CHUNK, _CHUNK)], sem))
            copies.append(pltpu.async_copy(
                table_hbm.at[src_o],
                rows_v.at[pl.ds(_BW + c * _CHUNK, _CHUNK)], sem))
        for cp in copies:
            cp.wait()
        pltpu.sync_copy(rows_v.at[pl.ds(0, _BW)],
                        out_hbm.at[j, pl.ds(b0, _BW), pl.ds(0, _D)])
        pltpu.sync_copy(rows_v.at[pl.ds(_BW, _BW)],
                        out_hbm.at[j, pl.ds(b0, _BW), pl.ds(_D, _D)])
        return carry

    lax.fori_loop(0, n_pairs, pair, 0)


def _sc_gather(table_lin, idxt):
    hist, batch = idxt.shape
    n_pairs = hist // 2
    mesh = plsc.VectorSubcoreMesh(core_axis_name="c", subcore_axis_name="s")
    kern = pl.kernel(
        functools.partial(_gather_body, n_pairs),
        mesh=mesh,
        compiler_params=pltpu.CompilerParams(use_tc_tiling_on_sc=False),
        out_type=jax.ShapeDtypeStruct((n_pairs, batch, 2 * _D), jnp.float32),
        scratch_types=[
            pltpu.VMEM((hist, _BW), jnp.int32),
            pltpu.VMEM((2 * _BW, _D), jnp.float32),
            pltpu.SemaphoreType.DMA,
        ],
    )
    return kern(table_lin, idxt)


def _unpack_body(x_ref, o_ref):
    xt = x_ref[0].T
    o_ref[0] = xt[:_D, :]
    o_ref[1] = xt[_D:, :]


def _tc_unpack(emb3, bl):
    n_pairs, batch, _ = emb3.shape
    return pl.pallas_call(
        _unpack_body,
        grid=(n_pairs, batch // bl),
        in_specs=[pl.BlockSpec((1, bl, 2 * _D), lambda j, i: (j, i, 0))],
        out_specs=pl.BlockSpec((2, _D, bl), lambda j, i: (j, 0, i)),
        out_shape=jax.ShapeDtypeStruct((2 * n_pairs, _D, batch), jnp.float32),
    )(emb3)


def kernel(element, table, W, b):
    batch, hist = element.shape
    el = element.astype(jnp.int32)
    i = el // (2 * _PB)
    q = el % (2 * _PB)
    idx_phys = i * (2 * _PB) + 2 * (q % _PB) + q // _PB
    idxt = idx_phys.T
    p2 = _tc_transform(table.T, W.T, b.reshape(1, _D))
    p2v = p2.reshape(p2.shape[0] * 2, _D)
    emb3 = _sc_gather(p2v, idxt)
    y3 = _tc_unpack(emb3, 8192)
    return y3.transpose(2, 0, 1)


# speedup vs baseline: 4.4230x; 1.0353x over previous
import functools

import jax
import jax.numpy as jnp
from jax import lax
from jax.experimental import pallas as pl
from jax.experimental.pallas import tpu as pltpu
from jax.experimental.pallas import tpu_sc as plsc

_D = 64
_PB = 16384
_CHUNK = 128
_BW = 512


def _transform_body(x_ref, wt_ref, b_ref, o_ref):
    x = x_ref[...]
    for half in range(2):
        xt = x[:, half * _PB:(half + 1) * _PB].T
        y = jnp.dot(xt, wt_ref[...], preferred_element_type=jnp.float32)
        y = jnp.maximum(y + b_ref[...], 0.0)
        o_ref[:, half * _D:(half + 1) * _D] = y


def _tc_transform(table_t, wt, b2d):
    v, n = table_t.shape
    grid_n = (n + 2 * _PB - 1) // (2 * _PB)
    return pl.pallas_call(
        _transform_body,
        grid=(grid_n,),
        in_specs=[
            pl.BlockSpec((_D, 2 * _PB), lambda i: (0, i)),
            pl.BlockSpec((_D, _D), lambda i: (0, 0)),
            pl.BlockSpec((1, _D), lambda i: (0, 0)),
        ],
        out_specs=pl.BlockSpec((_PB, 2 * _D), lambda i: (i, 0)),
        out_shape=jax.ShapeDtypeStruct((grid_n * _PB, 2 * _D), jnp.float32),
    )(table_t, wt, b2d)


def _gather_body(n_pairs, table_hbm, idxt_hbm, out_hbm, idx_v, rows_v, sem):
    info = plsc.get_sparse_core_info()
    nc = info.num_cores
    wid = lax.axis_index("s") * nc + lax.axis_index("c")
    b0 = wid * _BW
    pltpu.sync_copy(idxt_hbm.at[:, pl.ds(b0, _BW)], idx_v)

    def pair(j, carry):
        copies = []
        for c in range(_BW // _CHUNK):
            src_e = idx_v.at[2 * j, pl.ds(c * _CHUNK, _CHUNK)]
            src_o = idx_v.at[2 * j + 1, pl.ds(c * _CHUNK, _CHUNK)]
            copies.append(pltpu.async_copy(
                table_hbm.at[src_e], rows_v.at[pl.ds(c * _CHUNK, _CHUNK)], sem))
            copies.append(pltpu.async_copy(
                table_hbm.at[src_o],
                rows_v.at[pl.ds(_BW + c * _CHUNK, _CHUNK)], sem))
        for cp in copies:
            cp.wait()
        pltpu.sync_copy(rows_v.at[pl.ds(0, _BW)],
                        out_hbm.at[j, pl.ds(b0, _BW), pl.ds(0, _D)])
        pltpu.sync_copy(rows_v.at[pl.ds(_BW, _BW)],
                        out_hbm.at[j, pl.ds(b0, _BW), pl.ds(_D, _D)])
        return carry

    lax.fori_loop(0, n_pairs, pair, 0)


def _sc_gather(table_lin, idxt):
    hist, batch = idxt.shape
    n_pairs = hist // 2
    mesh = plsc.VectorSubcoreMesh(core_axis_name="c", subcore_axis_name="s")
    kern = pl.kernel(
        functools.partial(_gather_body, n_pairs),
        mesh=mesh,
        compiler_params=pltpu.CompilerParams(use_tc_tiling_on_sc=False),
        out_type=jax.ShapeDtypeStruct((n_pairs, batch, 2 * _D), jnp.float32),
        scratch_types=[
            pltpu.VMEM((hist, _BW), jnp.int32),
            pltpu.VMEM((2 * _BW, _D), jnp.float32),
            pltpu.SemaphoreType.DMA,
        ],
    )
    return kern(table_lin, idxt)


def _unpack_body(x_ref, o_ref):
    xt = x_ref[0].T
    o_ref[0] = xt[:_D, :]
    o_ref[1] = xt[_D:, :]


def _tc_unpack(emb3, bl):
    n_pairs, batch, _ = emb3.shape
    return pl.pallas_call(
        _unpack_body,
        grid=(n_pairs, batch // bl),
        in_specs=[pl.BlockSpec((1, bl, 2 * _D), lambda j, i: (j, i, 0))],
        out_specs=pl.BlockSpec((2, _D, bl), lambda j, i: (j, 0, i)),
        out_shape=jax.ShapeDtypeStruct((2 * n_pairs, _D, batch), jnp.float32),
    )(emb3)


def kernel(element, table, W, b):
    batch, hist = element.shape
    el = element.astype(jnp.int32)
    i = el // (2 * _PB)
    q = el % (2 * _PB)
    idx_phys = i * (2 * _PB) + 2 * (q % _PB) + q // _PB
    idxt = idx_phys.T
    p2 = _tc_transform(table.T, W.T, b.reshape(1, _D))
    p2v = p2.reshape(p2.shape[0] * 2, _D)
    emb3 = _sc_gather(p2v, idxt)
    y3 = _tc_unpack(emb3, 16384)
    return y3.transpose(2, 0, 1)
